# baseline (speedup 1.0000x reference)
import functools

import jax
import jax.numpy as jnp
from jax import lax
from jax.experimental import pallas as pl
from jax.experimental.pallas import tpu as pltpu
from jax.experimental.pallas import tpu_sc as plsc

B, NF, Fm, H, NC = 8, 4096, 8, 128, 512
FmH = Fm * H
NFB = 4
BLK = NF // NFB

NCORES, NSUB, LANES = 2, 16, 16
SEGT = NC // NSUB
CH = 8
NVR = FmH // LANES
BPC = B // NCORES
NFP = NF + LANES


def _scores_body(x_ref, w_ref, o_ref):
    xb = x_ref[0]
    res = jax.lax.dot_general(xb, w_ref[...], (((1,), (0,)), ((), ())),
                              preferred_element_type=jnp.float32)
    o_ref[...] = res.reshape(1, 1, BLK)


def _weights_body(s_ref, seg_ref, o_ref, bounds_ref):
    s = s_ref[:, 0, :]
    m = jnp.max(s, axis=1, keepdims=True)
    ex = jnp.exp(s - m)
    seg = seg_ref[0, 0]
    cols = jax.lax.broadcasted_iota(jnp.int32, (NF, NC), 1)
    onehot = (cols == seg[:, None]).astype(jnp.float32)
    denom = jax.lax.dot_general(ex, onehot, (((1,), (0,)), ((), ())),
                                preferred_element_type=jnp.float32)
    denom_g = jax.lax.dot_general(denom, onehot, (((1,), (1,)), ((), ())),
                                  preferred_element_type=jnp.float32)
    o_ref[...] = (ex / denom_g).reshape(B, 1, NF)

    jmat = jax.lax.broadcasted_iota(jnp.int32, (NSUB * LANES, NF), 0)
    t_idx = jmat // LANES
    l_idx = jmat % LANES
    thr = jnp.where(l_idx <= 1,
                    (t_idx + (l_idx == 1).astype(jnp.int32)) * SEGT, 0)
    cmp = (seg[None, :] < thr).astype(jnp.float32)
    ones = jnp.ones((NF, 1), jnp.float32)
    counts = jax.lax.dot_general(cmp, ones, (((1,), (0,)), ((), ())),
                                 preferred_element_type=jnp.float32)
    bounds_ref[...] = counts.astype(jnp.int32)


def _pool_sc_body(x_hbm, w_hbm, seg_hbm, bounds_hbm, out_hbm,
                  bounds_v, seg_v, w_v, buf, acc):
    cid = lax.axis_index("c")
    sid = lax.axis_index("s")
    lane_iota = lax.broadcasted_iota(jnp.int32, (LANES,), 0)

    pltpu.sync_copy(bounds_hbm, bounds_v)
    pltpu.sync_copy(seg_hbm, seg_v)
    bvec = bounds_v[sid]
    start = jnp.sum(jnp.where(lane_iota == 0, bvec, 0))
    end = jnp.sum(jnp.where(lane_iota == 1, bvec, 0))
    start_al = (start // CH) * CH
    nch = (end - start_al + (CH - 1)) // CH
    seg_base = sid * SEGT

    @pl.loop(0, BPC)
    def _(k):
        b = k * NCORES + cid

        @pl.loop(0, SEGT)
        def _(r):
            for v in range(NVR):
                acc[r, pl.ds(v * LANES, LANES)] = jnp.zeros((LANES,),
                                                            jnp.float32)

        pltpu.sync_copy(w_hbm.at[b], w_v)

        @pl.loop(0, nch)
        def _(jc):
            rc = pl.multiple_of(start_al + jc * CH, CH)
            pltpu.sync_copy(x_hbm.at[b, pl.ds(rc, CH)], buf)
            segvec = seg_v[pl.ds(rc, LANES)] - seg_base
            wvec = w_v[pl.ds(rc, LANES)]
            for jj in range(CH):
                row = rc + jj
                valid = (row >= start) & (row < end)
                jmask = lane_iota == jj
                seg_j = jnp.sum(jnp.where(jmask, segvec, 0))
                w_j = jnp.sum(jnp.where(jmask, wvec, 0.0))

                @pl.when(valid)
                def _():
                    wb = jnp.full((LANES,), w_j, jnp.float32)
                    for v in range(NVR):
                        sl = pl.ds(v * LANES, LANES)
                        plsc.addupdate(acc.at[seg_j, sl], buf[jj, sl] * wb)

        pltpu.sync_copy(acc, out_hbm.at[b, pl.ds(seg_base, SEGT)])


_pool_sc = functools.partial(
    pl.kernel,
    out_type=jax.ShapeDtypeStruct((B, NC, FmH), jnp.float32),
    mesh=plsc.VectorSubcoreMesh(core_axis_name="c", subcore_axis_name="s"),
    compiler_params=pltpu.CompilerParams(needs_layout_passes=False),
    scratch_types=[
        pltpu.VMEM((NSUB, LANES), jnp.int32),
        pltpu.VMEM((NFP,), jnp.int32),
        pltpu.VMEM((NFP,), jnp.float32),
        pltpu.VMEM((CH, FmH), jnp.float32),
        pltpu.VMEM((SEGT, FmH), jnp.float32),
    ],
)(_pool_sc_body)


@jax.jit
def kernel(x, segment_ids, W, b):
    del b
    xm = x.reshape(B, NF, FmH)
    seg = segment_ids.astype(jnp.int32)
    seg2d = seg.reshape(1, 1, NF)
    seg_pad = jnp.concatenate([seg, jnp.zeros((LANES,), jnp.int32)])
    wfull = (jnp.tile(W[:, 0], Fm) / Fm).reshape(FmH, 1)

    scores = pl.pallas_call(
        _scores_body,
        grid=(B, NFB),
        in_specs=[
            pl.BlockSpec((1, BLK, FmH), lambda bi, ni: (bi, ni, 0)),
            pl.BlockSpec((FmH, 1), lambda bi, ni: (0, 0)),
        ],
        out_specs=pl.BlockSpec((1, 1, BLK), lambda bi, ni: (bi, 0, ni)),
        out_shape=jax.ShapeDtypeStruct((B, 1, NF), jnp.float32),
    )(xm, wfull)

    w, bounds = pl.pallas_call(
        _weights_body,
        in_specs=[
            pl.BlockSpec((B, 1, NF), lambda: (0, 0, 0)),
            pl.BlockSpec((1, 1, NF), lambda: (0, 0, 0)),
        ],
        out_specs=[
            pl.BlockSpec((B, 1, NF), lambda: (0, 0, 0)),
            pl.BlockSpec((NSUB * LANES, 1), lambda: (0, 0)),
        ],
        out_shape=[
            jax.ShapeDtypeStruct((B, 1, NF), jnp.float32),
            jax.ShapeDtypeStruct((NSUB * LANES, 1), jnp.int32),
        ],
    )(scores, seg2d)
    bounds = bounds.reshape(NSUB, LANES)

    w_pad = jnp.concatenate(
        [w[:, 0, :], jnp.zeros((B, LANES), jnp.float32)], axis=1)
    pooled = _pool_sc(xm, w_pad, seg_pad, bounds)
    return pooled.reshape(B, NC, Fm, H)


# speedup vs baseline: 1.0257x; 1.0257x over previous
import functools

import jax
import jax.numpy as jnp
from jax import lax
from jax.experimental import pallas as pl
from jax.experimental.pallas import tpu as pltpu
from jax.experimental.pallas import tpu_sc as plsc

B, NF, Fm, H, NC = 8, 4096, 8, 128, 512
FmH = Fm * H
NFB = 4
BLK = NF // NFB

NCORES, NSUB, LANES = 2, 16, 16
SEGT = NC // NSUB
CH = 8
NVR = FmH // LANES
BPC = B // NCORES
MW = 2 * LANES


def _scores_body(x_ref, w_ref, o_ref):
    xb = x_ref[0]
    res = jax.lax.dot_general(xb, w_ref[...], (((1,), (0,)), ((), ())),
                              preferred_element_type=jnp.float32)
    o_ref[...] = res.reshape(1, 1, BLK)


def _weights_body(s_ref, seg_ref, o_ref, bounds_ref):
    s = s_ref[:, 0, :]
    m = jnp.max(s, axis=1, keepdims=True)
    ex = jnp.exp(s - m)
    seg = seg_ref[0, 0]
    cols = jax.lax.broadcasted_iota(jnp.int32, (NF, NC), 1)
    onehot = (cols == seg[:, None]).astype(jnp.float32)
    denom = jax.lax.dot_general(ex, onehot, (((1,), (0,)), ((), ())),
                                preferred_element_type=jnp.float32)
    denom_g = jax.lax.dot_general(denom, onehot, (((1,), (1,)), ((), ())),
                                  preferred_element_type=jnp.float32)
    o_ref[...] = (ex / denom_g).reshape(B, 1, NF)

    jmat = jax.lax.broadcasted_iota(jnp.int32, (NSUB * LANES, NF), 0)
    t_idx = jmat // LANES
    l_idx = jmat % LANES
    thr = jnp.where(l_idx <= 1,
                    (t_idx + (l_idx == 1).astype(jnp.int32)) * SEGT, 0)
    cmp = (seg[None, :] < thr).astype(jnp.float32)
    ones = jnp.ones((NF, 1), jnp.float32)
    counts = jax.lax.dot_general(cmp, ones, (((1,), (0,)), ((), ())),
                                 preferred_element_type=jnp.float32)
    bounds_ref[...] = counts.astype(jnp.int32)


def _pool_sc_body(x_hbm, meta_hbm, bounds_hbm, out_hbm,
                  bounds_v, buf, mbuf, acc,
                  sem_x0, sem_x1, sem_m0, sem_m1):
    cid = lax.axis_index("c")
    sid = lax.axis_index("s")
    lane_iota = lax.broadcasted_iota(jnp.int32, (LANES,), 0)

    pltpu.sync_copy(bounds_hbm, bounds_v)
    bvec = bounds_v[sid]
    start = jnp.sum(jnp.where(lane_iota == 0, bvec, 0))
    end = jnp.sum(jnp.where(lane_iota == 1, bvec, 0))
    start_al = (start // CH) * CH
    nch = (end - start_al + (CH - 1)) // CH
    startv = jnp.full((LANES,), start, jnp.int32)
    endv = jnp.full((LANES,), end, jnp.int32)
    sems_x = (sem_x0, sem_x1)
    sems_m = (sem_m0, sem_m1)

    def chunk_base(jc):
        return pl.multiple_of(start_al + jc * CH, CH)

    def start_dma(b, jc, slot):
        rc = chunk_base(jc)
        pltpu.async_copy(x_hbm.at[b, pl.ds(rc, CH)], buf.at[slot],
                         sems_x[slot])
        pltpu.async_copy(meta_hbm.at[b, pl.ds(rc, CH)], mbuf.at[slot],
                         sems_m[slot])

    def process(b, jc, slot):
        pltpu.make_async_copy(x_hbm.at[b, pl.ds(0, CH)], buf.at[slot],
                              sems_x[slot]).wait()
        pltpu.make_async_copy(meta_hbm.at[b, pl.ds(0, CH)], mbuf.at[slot],
                              sems_m[slot]).wait()
        rc = chunk_base(jc)
        rcv = jnp.full((LANES,), rc, jnp.int32)
        for jj in range(CH):
            rowv = rcv + jj
            maskv = (rowv >= startv) & (rowv < endv)
            addrv = mbuf[slot, jj, pl.ds(0, LANES)]
            wv = plsc.bitcast(mbuf[slot, jj, pl.ds(LANES, LANES)],
                              jnp.float32)
            for v in range(NVR):
                vals = buf[slot, jj, pl.ds(v * LANES, LANES)] * wv
                plsc.addupdate_scatter(acc, [addrv + (v * LANES)], vals,
                                       mask=maskv)

    @pl.loop(0, BPC)
    def _(k):
        b = k * NCORES + cid

        @pl.loop(0, SEGT * NVR // 16)
        def _(r):
            for u in range(16):
                acc[pl.ds((r * 16 + u) * LANES, LANES)] = jnp.zeros(
                    (LANES,), jnp.float32)

        @pl.when(nch > 0)
        def _():
            start_dma(b, 0, 0)

        @pl.loop(0, nch)
        def _(jc):
            @pl.when(jc + 1 < nch)
            def _():
                @pl.when(jc % 2 == 0)
                def _():
                    start_dma(b, jc + 1, 1)

                @pl.when(jc % 2 == 1)
                def _():
                    start_dma(b, jc + 1, 0)

            @pl.when(jc % 2 == 0)
            def _():
                process(b, jc, 0)

            @pl.when(jc % 2 == 1)
            def _():
                process(b, jc, 1)

        pltpu.sync_copy(acc, out_hbm.at[b, pl.ds(sid * SEGT * FmH,
                                                 SEGT * FmH)])


_pool_sc = functools.partial(
    pl.kernel,
    out_type=jax.ShapeDtypeStruct((B, NC * FmH), jnp.float32),
    mesh=plsc.VectorSubcoreMesh(core_axis_name="c", subcore_axis_name="s"),
    compiler_params=pltpu.CompilerParams(needs_layout_passes=False),
    scratch_types=[
        pltpu.VMEM((NSUB, LANES), jnp.int32),
        pltpu.VMEM((2, CH, FmH), jnp.float32),
        pltpu.VMEM((2, CH, MW), jnp.int32),
        pltpu.VMEM((SEGT * FmH,), jnp.float32),
        pltpu.SemaphoreType.DMA,
        pltpu.SemaphoreType.DMA,
        pltpu.SemaphoreType.DMA,
        pltpu.SemaphoreType.DMA,
    ],
)(_pool_sc_body)


@jax.jit
def kernel(x, segment_ids, W, b):
    del b
    xm = x.reshape(B, NF, FmH)
    seg = segment_ids.astype(jnp.int32)
    seg2d = seg.reshape(1, 1, NF)
    wfull = (jnp.tile(W[:, 0], Fm) / Fm).reshape(FmH, 1)

    scores = pl.pallas_call(
        _scores_body,
        grid=(B, NFB),
        in_specs=[
            pl.BlockSpec((1, BLK, FmH), lambda bi, ni: (bi, ni, 0)),
            pl.BlockSpec((FmH, 1), lambda bi, ni: (0, 0)),
        ],
        out_specs=pl.BlockSpec((1, 1, BLK), lambda bi, ni: (bi, 0, ni)),
        out_shape=jax.ShapeDtypeStruct((B, 1, NF), jnp.float32),
    )(xm, wfull)

    w, bounds = pl.pallas_call(
        _weights_body,
        in_specs=[
            pl.BlockSpec((B, 1, NF), lambda: (0, 0, 0)),
            pl.BlockSpec((1, 1, NF), lambda: (0, 0, 0)),
        ],
        out_specs=[
            pl.BlockSpec((B, 1, NF), lambda: (0, 0, 0)),
            pl.BlockSpec((NSUB * LANES, 1), lambda: (0, 0)),
        ],
        out_shape=[
            jax.ShapeDtypeStruct((B, 1, NF), jnp.float32),
            jax.ShapeDtypeStruct((NSUB * LANES, 1), jnp.int32),
        ],
    )(scores, seg2d)
    bounds = bounds.reshape(NSUB, LANES)

    lanes = jnp.arange(LANES, dtype=jnp.int32)
    addr = ((seg % SEGT) * FmH)[:, None] + lanes[None, :]
    addr_b = jnp.broadcast_to(addr[None], (B, NF, LANES))
    wbits = lax.bitcast_convert_type(
        jnp.broadcast_to(w[:, 0, :, None], (B, NF, LANES)), jnp.int32)
    meta = jnp.concatenate([addr_b, wbits], axis=2)

    pooled = _pool_sc(xm, meta, bounds)
    return pooled.reshape(B, NC, Fm, H)


# speedup vs baseline: 3.0064x; 2.9311x over previous
import jax
import jax.numpy as jnp
from jax.experimental import pallas as pl
from jax.experimental.pallas import tpu as pltpu

B, NF, Fm, H, NC = 8, 4096, 8, 128, 512
FmH = Fm * H


def _fused_body(x_ref, w_ref, seg_ref, o_ref):
    xb = x_ref[0]
    scores = jax.lax.dot_general(xb, w_ref[...], (((1,), (0,)), ((), ())),
                                 preferred_element_type=jnp.float32)
    m = jnp.max(scores)
    ex = jnp.exp(scores - m)

    seg = seg_ref[0, 0]
    cols = jax.lax.broadcasted_iota(jnp.int32, (NF, NC), 1)
    onehot = (cols == seg[:, None]).astype(jnp.float32)

    denom = jax.lax.dot_general(ex, onehot, (((0,), (0,)), ((), ())),
                                preferred_element_type=jnp.float32)
    denom_g = jax.lax.dot_general(onehot, denom, (((1,), (1,)), ((), ())),
                                  preferred_element_type=jnp.float32)
    w_row = ex / denom_g

    a = (onehot * w_row).astype(jnp.bfloat16)
    pooled = jax.lax.dot_general(a, xb.astype(jnp.bfloat16),
                                 (((0,), (0,)), ((), ())),
                                 preferred_element_type=jnp.float32)
    o_ref[0] = pooled


@jax.jit
def kernel(x, segment_ids, W, b):
    del b
    xm = x.reshape(B, NF, FmH)
    seg2d = segment_ids.astype(jnp.int32).reshape(1, 1, NF)
    wfull = (jnp.tile(W[:, 0], Fm) / Fm).reshape(FmH, 1)

    pooled = pl.pallas_call(
        _fused_body,
        grid=(B,),
        in_specs=[
            pl.BlockSpec((1, NF, FmH), lambda bi: (bi, 0, 0)),
            pl.BlockSpec((FmH, 1), lambda bi: (0, 0)),
            pl.BlockSpec((1, 1, NF), lambda bi: (0, 0, 0)),
        ],
        out_specs=pl.BlockSpec((1, NC, FmH), lambda bi: (bi, 0, 0)),
        out_shape=jax.ShapeDtypeStruct((B, NC, FmH), jnp.float32),
        compiler_params=pltpu.CompilerParams(
            vmem_limit_bytes=100 * 1024 * 1024),
    )(xm, wfull, seg2d)

    return pooled.reshape(B, NC, Fm, H)


# speedup vs baseline: 3.4982x; 1.1636x over previous
import jax
import jax.numpy as jnp
from jax.experimental import pallas as pl
from jax.experimental.pallas import tpu as pltpu

B, NF, Fm, H, NC = 8, 4096, 8, 128, 512
FmH = Fm * H


def _fused_body(x_ref, w_ref, seg_ref, o_ref):
    xb = x_ref[0]
    scores = jnp.sum(xb * w_ref[...].reshape(1, FmH), axis=1,
                     keepdims=True)
    m = jnp.max(scores)
    ex = jnp.exp(scores - m)

    seg = seg_ref[0, 0]
    cols = jax.lax.broadcasted_iota(jnp.int32, (NF, NC), 1)
    onehot = (cols == seg[:, None]).astype(jnp.float32)

    denom = jnp.sum(onehot * ex, axis=0, keepdims=True)
    denom_g = jnp.sum(onehot * denom, axis=1, keepdims=True)
    w_row = ex / denom_g

    a = (onehot * w_row).astype(jnp.bfloat16)
    pooled = jax.lax.dot_general(a, xb.astype(jnp.bfloat16),
                                 (((0,), (0,)), ((), ())),
                                 preferred_element_type=jnp.float32)
    o_ref[0] = pooled


@jax.jit
def kernel(x, segment_ids, W, b):
    del b
    xm = x.reshape(B, NF, FmH)
    seg2d = segment_ids.astype(jnp.int32).reshape(1, 1, NF)
    wfull = (jnp.tile(W[:, 0], Fm) / Fm).reshape(FmH, 1)

    pooled = pl.pallas_call(
        _fused_body,
        grid=(B,),
        in_specs=[
            pl.BlockSpec((1, NF, FmH), lambda bi: (bi, 0, 0)),
            pl.BlockSpec((FmH, 1), lambda bi: (0, 0)),
            pl.BlockSpec((1, 1, NF), lambda bi: (0, 0, 0)),
        ],
        out_specs=pl.BlockSpec((1, NC, FmH), lambda bi: (bi, 0, 0)),
        out_shape=jax.ShapeDtypeStruct((B, NC, FmH), jnp.float32),
        compiler_params=pltpu.CompilerParams(
            vmem_limit_bytes=100 * 1024 * 1024),
    )(xm, wfull, seg2d)

    return pooled.reshape(B, NC, Fm, H)


# speedup vs baseline: 3.5214x; 1.0066x over previous
import jax
import jax.numpy as jnp
from jax.experimental import pallas as pl
from jax.experimental.pallas import tpu as pltpu

B, NF, Fm, H, NC = 8, 4096, 8, 128, 512
FmH = Fm * H


def _fused_body(x_ref, w_ref, seg_ref, o_ref):
    xb = x_ref[0]
    scores = jnp.sum(xb * w_ref[...].reshape(1, FmH), axis=1,
                     keepdims=True)
    m = jnp.max(scores)
    ex = jnp.exp(scores - m)

    seg = seg_ref[0, 0]
    cols = jax.lax.broadcasted_iota(jnp.int32, (NF, NC), 1)
    onehot = (cols == seg[:, None]).astype(jnp.float32)

    a = (onehot * ex).astype(jnp.bfloat16)
    xaug = jnp.concatenate(
        [xb.astype(jnp.bfloat16),
         jnp.ones((NF, 128), jnp.bfloat16)], axis=1)
    pooled_u = jax.lax.dot_general(a, xaug, (((0,), (0,)), ((), ())),
                                   preferred_element_type=jnp.float32)
    denom = pooled_u[:, FmH:FmH + 1]
    inv = 1.0 / jnp.where(denom == 0.0, 1.0, denom)
    o_ref[0] = pooled_u[:, :FmH] * inv


@jax.jit
def kernel(x, segment_ids, W, b):
    del b
    xm = x.reshape(B, NF, FmH)
    seg2d = segment_ids.astype(jnp.int32).reshape(1, 1, NF)
    wfull = (jnp.tile(W[:, 0], Fm) / Fm).reshape(FmH, 1)

    pooled = pl.pallas_call(
        _fused_body,
        grid=(B,),
        in_specs=[
            pl.BlockSpec((1, NF, FmH), lambda bi: (bi, 0, 0)),
            pl.BlockSpec((FmH, 1), lambda bi: (0, 0)),
            pl.BlockSpec((1, 1, NF), lambda bi: (0, 0, 0)),
        ],
        out_specs=pl.BlockSpec((1, NC, FmH), lambda bi: (bi, 0, 0)),
        out_shape=jax.ShapeDtypeStruct((B, NC, FmH), jnp.float32),
        compiler_params=pltpu.CompilerParams(
            vmem_limit_bytes=100 * 1024 * 1024),
    )(xm, wfull, seg2d)

    return pooled.reshape(B, NC, Fm, H)
